# baseline (speedup 1.0000x reference)
import functools

import jax
import jax.numpy as jnp
from jax import lax
from jax.experimental import pallas as pl
from jax.experimental.pallas import tpu as pltpu
from jax.experimental.pallas import tpu_sc as plsc

_EPS = 1e-6
_NSEG = 256
_BINS = 512


@functools.cache
def _make_sc_counts(n_pad: int, n_per_w: int, nw: int):
    nvr = n_per_w // 16
    mesh = plsc.VectorSubcoreMesh(
        core_axis_name="c", subcore_axis_name="s", num_cores=2
    )

    @functools.partial(
        pl.kernel,
        out_type=jax.ShapeDtypeStruct((2 * _NSEG,), jnp.float32),
        mesh=mesh,
        compiler_params=pltpu.CompilerParams(needs_layout_passes=False),
        scratch_types=[
            pltpu.VMEM((n_per_w,), jnp.int32),
            pltpu.VMEM((_BINS,), jnp.float32),
            pltpu.VMEM((16, _BINS), jnp.float32),
            pltpu.VMEM((16,), jnp.float32),
            pltpu.VMEM_SHARED((16, _BINS), jnp.float32),
        ],
    )
    def sc_k(ids_hbm, out_hbm, ids_v, bins_v, all_v, stage_v, sh_bins):
        cid = lax.axis_index("c")
        sid = lax.axis_index("s")
        wid = cid * 16 + sid
        base = wid * n_per_w
        pltpu.sync_copy(ids_hbm.at[pl.ds(base, n_per_w)], ids_v)

        zero16 = jnp.zeros((16,), jnp.float32)
        ones16 = jnp.ones((16,), jnp.float32)

        def zero_body(j, carry):
            bins_v[pl.ds(j * 16, 16)] = zero16
            return carry

        lax.fori_loop(0, _BINS // 16, zero_body, 0)

        def acc_body(j, carry):
            idv = ids_v[pl.ds(j * 16, 16)]
            plsc.addupdate_scatter(bins_v, [idv], ones16)
            return carry

        lax.fori_loop(0, nvr, acc_body, 0)

        pltpu.sync_copy(bins_v, sh_bins.at[sid])
        plsc.subcore_barrier()
        pltpu.sync_copy(sh_bins, all_v)

        off = sid * 16
        acc = jnp.zeros((16,), jnp.float32)
        for r in range(16):
            acc = acc + all_v[r, pl.ds(off, 16)]
        stage_v[...] = acc
        pltpu.sync_copy(stage_v, out_hbm.at[pl.ds(cid * _NSEG + off, 16)])

    return sc_k


def _partial_body(x_ref, ids_ref, o_ref):
    xb = x_ref[...]
    rowsq = jnp.sum(xb * xb, axis=1, keepdims=True)
    ids = ids_ref[0, 0, :]
    iota = lax.broadcasted_iota(jnp.int32, (1, _NSEG), 1)
    onehot = (ids[:, None] == iota).astype(jnp.float32)
    o_ref[...] = lax.dot_general(
        rowsq, onehot, (((0,), (0,)), ((), ())),
        preferred_element_type=jnp.float32,
    )


def _make_apply_body(nblk):
    def _apply_body(x_ref, ids_ref, p_ref, cnt_ref, w_ref, o_ref, scale_ref):
        i = pl.program_id(0)

        @pl.when(i == 0)
        def _():
            tot = p_ref[:, 0:_NSEG]
            for b in range(1, nblk):
                tot = tot + p_ref[:, b * _NSEG:(b + 1) * _NSEG]
            cnt = jnp.maximum(
                cnt_ref[:, 0:_NSEG] + cnt_ref[:, _NSEG:2 * _NSEG], 1.0
            )
            scale_ref[...] = lax.rsqrt(
                tot / (cnt * float(x_ref.shape[1])) + _EPS
            )

        ids = ids_ref[0, 0, :]
        iota = lax.broadcasted_iota(jnp.int32, (1, _NSEG), 1)
        onehot = (ids[:, None] == iota).astype(jnp.float32)
        rowscale = lax.dot_general(
            onehot, scale_ref[...], (((1,), (1,)), ((), ())),
            preferred_element_type=jnp.float32,
        )
        o_ref[...] = x_ref[...] * (w_ref[...] * rowscale)

    return _apply_body


def kernel(x, batch, weight):
    n, feat = x.shape
    ids = batch.astype(jnp.int32)

    nw = 32
    n_per_w = ((n + nw * 16 - 1) // (nw * 16)) * 16
    n_pad = nw * n_per_w
    ids_p = jnp.pad(ids, (0, n_pad - n), constant_values=_NSEG)
    counts = _make_sc_counts(n_pad, n_per_w, nw)(ids_p)

    nblk = 10
    rb = n // nblk
    ids3 = ids.reshape(nblk, 1, rb)
    w2 = weight.reshape(1, feat).astype(jnp.float32)
    c2 = counts.reshape(1, 2 * _NSEG)

    partials = pl.pallas_call(
        _partial_body,
        grid=(nblk,),
        in_specs=[
            pl.BlockSpec((rb, feat), lambda i: (i, 0)),
            pl.BlockSpec((1, 1, rb), lambda i: (i, 0, 0)),
        ],
        out_specs=pl.BlockSpec((1, _NSEG), lambda i: (0, i)),
        out_shape=jax.ShapeDtypeStruct((1, nblk * _NSEG), jnp.float32),
    )(x, ids3)

    out = pl.pallas_call(
        _make_apply_body(nblk),
        grid=(nblk,),
        in_specs=[
            pl.BlockSpec((rb, feat), lambda i: (i, 0)),
            pl.BlockSpec((1, 1, rb), lambda i: (i, 0, 0)),
            pl.BlockSpec((1, nblk * _NSEG), lambda i: (0, 0)),
            pl.BlockSpec((1, 2 * _NSEG), lambda i: (0, 0)),
            pl.BlockSpec((1, feat), lambda i: (0, 0)),
        ],
        out_specs=pl.BlockSpec((rb, feat), lambda i: (i, 0)),
        out_shape=jax.ShapeDtypeStruct((n, feat), x.dtype),
        scratch_shapes=[
            pltpu.VMEM((1, _NSEG), jnp.float32),
        ],
    )(x, ids3, partials, c2, w2)
    return out


# speedup vs baseline: 1.0155x; 1.0155x over previous
import functools

import jax
import jax.numpy as jnp
from jax import lax
from jax.experimental import pallas as pl
from jax.experimental.pallas import tpu as pltpu
from jax.experimental.pallas import tpu_sc as plsc

_EPS = 1e-6
_NSEG = 256
_BINS = 512


@functools.cache
def _make_sc_counts(n_pad: int, n_per_w: int, nw: int):
    nvr = n_per_w // 16
    mesh = plsc.VectorSubcoreMesh(
        core_axis_name="c", subcore_axis_name="s", num_cores=1
    )

    @functools.partial(
        pl.kernel,
        out_type=jax.ShapeDtypeStruct((_NSEG,), jnp.float32),
        mesh=mesh,
        compiler_params=pltpu.CompilerParams(needs_layout_passes=False),
        scratch_types=[
            pltpu.VMEM((n_per_w,), jnp.int32),
            pltpu.VMEM((_BINS,), jnp.float32),
            pltpu.VMEM((16, _BINS), jnp.float32),
            pltpu.VMEM((16,), jnp.float32),
            pltpu.VMEM_SHARED((16, _BINS), jnp.float32),
        ],
    )
    def sc_k(ids_hbm, out_hbm, ids_v, bins_v, all_v, stage_v, sh_bins):
        sid = lax.axis_index("s")
        base = sid * n_per_w
        pltpu.sync_copy(ids_hbm.at[pl.ds(base, n_per_w)], ids_v)

        zero16 = jnp.zeros((16,), jnp.float32)
        ones16 = jnp.ones((16,), jnp.float32)

        def zero_body(j, carry):
            bins_v[pl.ds(j * 16, 16)] = zero16
            return carry

        lax.fori_loop(0, _BINS // 16, zero_body, 0)

        def acc_body(j, carry):
            idv = ids_v[pl.ds(j * 16, 16)]
            plsc.addupdate_scatter(bins_v, [idv], ones16)
            return carry

        lax.fori_loop(0, nvr, acc_body, 0)

        pltpu.sync_copy(bins_v, sh_bins.at[sid])
        plsc.subcore_barrier()
        pltpu.sync_copy(sh_bins, all_v)

        off = sid * 16
        acc = jnp.zeros((16,), jnp.float32)
        for r in range(16):
            acc = acc + all_v[r, pl.ds(off, 16)]
        stage_v[...] = acc
        pltpu.sync_copy(stage_v, out_hbm.at[pl.ds(off, 16)])

    return sc_k


def _partial_body(x_ref, ids_ref, o_ref):
    xb = x_ref[...]
    rowsq = jnp.sum(xb * xb, axis=1, keepdims=True)
    ids = ids_ref[0, 0, :]
    iota = lax.broadcasted_iota(jnp.int32, (1, _NSEG), 1)
    onehot = (ids[:, None] == iota).astype(jnp.float32)
    o_ref[...] = lax.dot_general(
        rowsq, onehot, (((0,), (0,)), ((), ())),
        preferred_element_type=jnp.float32,
    )


def _make_apply_body(nblk):
    def _apply_body(x_ref, ids_ref, p_ref, cnt_ref, w_ref, o_ref, scale_ref):
        i = pl.program_id(0)

        @pl.when(i == 0)
        def _():
            tot = p_ref[:, 0:_NSEG]
            for b in range(1, nblk):
                tot = tot + p_ref[:, b * _NSEG:(b + 1) * _NSEG]
            cnt = jnp.maximum(cnt_ref[...], 1.0)
            scale_ref[...] = lax.rsqrt(
                tot / (cnt * float(x_ref.shape[1])) + _EPS
            )

        ids = ids_ref[0, 0, :]
        iota = lax.broadcasted_iota(jnp.int32, (1, _NSEG), 1)
        onehot = (ids[:, None] == iota).astype(jnp.float32)
        rowscale = lax.dot_general(
            onehot, scale_ref[...], (((1,), (1,)), ((), ())),
            preferred_element_type=jnp.float32,
        )
        o_ref[...] = x_ref[...] * (w_ref[...] * rowscale)

    return _apply_body


def kernel(x, batch, weight):
    n, feat = x.shape
    ids = batch.astype(jnp.int32)

    nw = 16
    n_per_w = ((n + nw * 16 - 1) // (nw * 16)) * 16
    n_pad = nw * n_per_w
    ids_p = jnp.pad(ids, (0, n_pad - n), constant_values=_NSEG)
    counts = _make_sc_counts(n_pad, n_per_w, nw)(ids_p)

    nblk = 10
    rb = n // nblk
    ids3 = ids.reshape(nblk, 1, rb)
    w2 = weight.reshape(1, feat).astype(jnp.float32)
    c2 = counts.reshape(1, _NSEG)

    partials = pl.pallas_call(
        _partial_body,
        grid=(nblk,),
        in_specs=[
            pl.BlockSpec((rb, feat), lambda i: (i, 0)),
            pl.BlockSpec((1, 1, rb), lambda i: (i, 0, 0)),
        ],
        out_specs=pl.BlockSpec((1, _NSEG), lambda i: (0, i)),
        out_shape=jax.ShapeDtypeStruct((1, nblk * _NSEG), jnp.float32),
    )(x, ids3)

    out = pl.pallas_call(
        _make_apply_body(nblk),
        grid=(nblk,),
        in_specs=[
            pl.BlockSpec((rb, feat), lambda i: (i, 0)),
            pl.BlockSpec((1, 1, rb), lambda i: (i, 0, 0)),
            pl.BlockSpec((1, nblk * _NSEG), lambda i: (0, 0)),
            pl.BlockSpec((1, _NSEG), lambda i: (0, 0)),
            pl.BlockSpec((1, feat), lambda i: (0, 0)),
        ],
        out_specs=pl.BlockSpec((rb, feat), lambda i: (i, 0)),
        out_shape=jax.ShapeDtypeStruct((n, feat), x.dtype),
        scratch_shapes=[
            pltpu.VMEM((1, _NSEG), jnp.float32),
        ],
    )(x, ids3, partials, c2, w2)
    return out
